# Initial kernel scaffold; baseline (speedup 1.0000x reference)
import jax
import jax.numpy as jnp
from jax.experimental import pallas as pl


def kernel(feat, W1, W2, W_cls, neigh_idx, nodes):
    raise NotImplementedError("write your pallas kernel here")


# speedup vs baseline: 8.3466x; 8.3466x over previous
import functools

import jax
import jax.numpy as jnp
from jax import lax
from jax.experimental import pallas as pl
from jax.experimental.pallas import tpu as pltpu
from jax.experimental.pallas import tpu_sc as plsc

NC = 2
NS = 16
NW = NC * NS
LANES = 16


def _sc_aggregate(feat, nbp, nodes_flat, rmap, cmap, S, G):
    N, D = feat.shape
    NF = nodes_flat.shape[0]
    PAD = nbp.shape[1]
    T = NF * S
    npw = NF // NW
    tpw = npw * S
    nchunk = tpw // G
    GS = G * S
    slices = []
    off = 0
    while off < GS:
        c = min(128, GS - off)
        slices.append((off, c))
        off += c

    mesh = plsc.VectorSubcoreMesh(core_axis_name="c", subcore_axis_name="s")

    @functools.partial(
        pl.kernel,
        mesh=mesh,
        compiler_params=pltpu.CompilerParams(
            needs_layout_passes=False, use_tc_tiling_on_sc=False),
        out_type=jax.ShapeDtypeStruct((T, D), jnp.float32),
        scratch_types=[
            pltpu.VMEM((npw,), jnp.int32),
            pltpu.VMEM((npw, PAD), jnp.int32),
            pltpu.VMEM((tpw,), jnp.int32),
            pltpu.VMEM((tpw, PAD), jnp.int32),
            pltpu.VMEM((2, GS), jnp.int32),
            pltpu.VMEM((2, GS, D), jnp.float32),
            pltpu.VMEM((2, G, D), jnp.float32),
            pltpu.VMEM((tpw,), jnp.int32),
            pltpu.VMEM((tpw,), jnp.int32),
            pltpu.SemaphoreType.DMA,
            pltpu.SemaphoreType.DMA,
            pltpu.SemaphoreType.DMA,
            pltpu.SemaphoreType.DMA,
        ],
    )
    def k(feat_hbm, nbp_hbm, nodes_hbm, rmap_hbm, cmap_hbm, out_hbm,
          nodes_v, nb2_v, fh_v, nb_v, fidx_v, rows_v, out_v,
          rmap_v, cmap_v,
          rsem0, rsem1, osem0, osem1):
        rsem = (rsem0, rsem1)
        osem = (osem0, osem1)
        wid = lax.axis_index("s") * NC + lax.axis_index("c")
        node_base = wid * npw
        tgt_base = wid * tpw

        pltpu.sync_copy(rmap_hbm, rmap_v)
        pltpu.sync_copy(cmap_hbm, cmap_v)
        pltpu.sync_copy(nodes_hbm.at[pl.ds(node_base, npw)], nodes_v)
        pltpu.async_copy(nbp_hbm.at[nodes_v], nb2_v, rsem0).wait()

        def build_fh(t, _):
            t16 = pl.multiple_of(t * LANES, LANES)
            r = rmap_v[pl.ds(t16, LANES)]
            c = cmap_v[pl.ds(t16, LANES)]
            fh_v[pl.ds(t16, LANES)] = plsc.load_gather(nb2_v, [r, c])
            return 0
        lax.fori_loop(0, tpw // LANES, build_fh, 0)

        hs = []
        for q in range(tpw // 128):
            hs.append(pltpu.async_copy(
                nbp_hbm.at[fh_v.at[pl.ds(128 * q, 128)]],
                nb_v.at[pl.ds(128 * q, 128)], rsem0))
        for h in hs:
            h.wait()

        def build_fidx(b, cidx):
            tb = cidx * G
            def bt(t, _):
                t16 = pl.multiple_of(t * LANES, LANES)
                g = rmap_v[pl.ds(t16, LANES)]
                c = cmap_v[pl.ds(t16, LANES)]
                fidx_v[b, pl.ds(t16, LANES)] = plsc.load_gather(
                    nb_v, [tb + g, c])
                return 0
            lax.fori_loop(0, GS // LANES, bt, 0)

        def fire_rows(b):
            for (o, c) in slices:
                pltpu.async_copy(
                    feat_hbm.at[fidx_v.at[b, pl.ds(o, c)]],
                    rows_v.at[b, pl.ds(o, c)], rsem[b])

        def wait_rows(b):
            for (o, c) in slices:
                pltpu.make_async_copy(
                    feat_hbm.at[fidx_v.at[b, pl.ds(o, c)]],
                    rows_v.at[b, pl.ds(o, c)], rsem[b]).wait()

        build_fidx(0, 0)
        fire_rows(0)
        build_fidx(1, 1)
        fire_rows(1)

        def chunk_iter(i, _):
            for b in range(2):
                cc = 2 * i + b
                wait_rows(b)

                @pl.when(i >= 1)
                def _():
                    pltpu.make_async_copy(
                        out_v.at[b], out_hbm.at[pl.ds(tgt_base, G)],
                        osem[b]).wait()

                def gacc(g, _):
                    base = g * S
                    for d in range(D // LANES):
                        col = pl.ds(d * LANES, LANES)
                        acc = rows_v[b, base, col]
                        for s2 in range(1, S):
                            acc = acc + rows_v[b, base + s2, col]
                        out_v[b, g, col] = acc
                    return 0
                lax.fori_loop(0, G, gacc, 0)

                pltpu.async_copy(
                    out_v.at[b],
                    out_hbm.at[pl.ds(tgt_base + cc * G, G)], osem[b])

                @pl.when(i < nchunk // 2 - 1)
                def _():
                    build_fidx(b, cc + 2)
                    fire_rows(b)
            return 0
        lax.fori_loop(0, nchunk // 2, chunk_iter, 0)

        for b in range(2):
            pltpu.make_async_copy(
                out_v.at[b], out_hbm.at[pl.ds(tgt_base, G)], osem[b]).wait()

    return k(feat, nbp, nodes_flat, rmap, cmap)


def _tc_tail(sum1, W1, W2, W_cls, S, blocks):
    T, D = sum1.shape
    E = W2.shape[0]
    C = W_cls.shape[0]
    rows = T // blocks
    grp = rows // S
    pairs = grp // 2
    Bout = blocks * pairs

    def body(x_ref, w1_ref, w2_ref, wc_ref, o_ref):
        x = x_ref[...]
        r = lax.dot_general(x, w1_ref[...], (((1,), (1,)), ((), ())),
                            preferred_element_type=jnp.float32)
        r = jnp.maximum(r, 0.0)
        p1 = (lax.broadcasted_iota(jnp.int32, (grp, rows), 0)
              == lax.broadcasted_iota(jnp.int32, (grp, rows), 1) // S
              ).astype(jnp.float32)
        t = lax.dot_general(p1, r, (((1,), (0,)), ((), ())),
                            preferred_element_type=jnp.float32)
        u = lax.dot_general(t, w2_ref[...], (((1,), (1,)), ((), ())),
                            preferred_element_type=jnp.float32)
        p2 = (lax.broadcasted_iota(jnp.int32, (pairs, grp), 0)
              == lax.broadcasted_iota(jnp.int32, (pairs, grp), 1) // 2
              ).astype(jnp.float32)
        v = lax.dot_general(p2, u, (((1,), (0,)), ((), ())),
                            preferred_element_type=jnp.float32)
        sc_ = lax.dot_general(v, wc_ref[...], (((1,), (1,)), ((), ())),
                              preferred_element_type=jnp.float32)
        o_ref[...] = sc_ * (1.0 / (S * S * 2.0))

    return pl.pallas_call(
        body,
        grid=(blocks,),
        in_specs=[
            pl.BlockSpec((rows, D), lambda i: (i, 0)),
            pl.BlockSpec((E, D), lambda i: (0, 0)),
            pl.BlockSpec((E, E), lambda i: (0, 0)),
            pl.BlockSpec((C, E), lambda i: (0, 0)),
        ],
        out_specs=pl.BlockSpec((pairs, C), lambda i: (i, 0)),
        out_shape=jax.ShapeDtypeStruct((Bout, C), jnp.float32),
    )(sum1, W1, W2, W_cls)


def kernel(feat, W1, W2, W_cls, neigh_idx, nodes):
    S = neigh_idx.shape[1]
    nbp = jnp.pad(neigh_idx.astype(jnp.int32), ((0, 0), (0, 16 - S)))
    nodes_flat = nodes.reshape(-1).astype(jnp.int32)
    tpw = nodes_flat.shape[0] * S // NW
    karr = jnp.arange(tpw, dtype=jnp.int32)
    sum1 = _sc_aggregate(feat, nbp, nodes_flat, karr // S, karr % S, S, 32)
    return _tc_tail(sum1, W1, W2, W_cls, S, 16)
